# baseline (speedup 1.0000x reference)
import functools

import jax
import jax.numpy as jnp
import numpy as np
from jax import lax
from jax.experimental import pallas as pl

BATCH = 16384
VOCAB = 1024
SNR = 10.0
_A = float(np.sqrt(10 ** (SNR / 10)))
_SCALE = (2.0 * _A) / (VOCAB - 1.0)

ROWS = 512

_KS0 = np.uint32(0)
_KS1 = np.uint32(42)
_KS2 = np.uint32(int(_KS0) ^ int(_KS1) ^ 0x1BD11BDA)
_R0 = (13, 15, 26, 6)
_R1 = (17, 29, 16, 24)
_TINY = np.float32(np.finfo(np.float32).tiny)


def _rotl(x, r):
    return lax.shift_left(x, jnp.uint32(r)) | lax.shift_right_logical(
        x, jnp.uint32(32 - r)
    )


def _round4(x0, x1, rots):
    for r in rots:
        x0 = x0 + x1
        x1 = _rotl(x1, r) ^ x0
    return x0, x1


def _threefry_fold(lin):
    x0 = jnp.full_like(lin, _KS0)
    x1 = lin + _KS1
    x0, x1 = _round4(x0, x1, _R0)
    x0, x1 = x0 + _KS1, x1 + (_KS2 + np.uint32(1))
    x0, x1 = _round4(x0, x1, _R1)
    x0, x1 = x0 + _KS2, x1 + (_KS0 + np.uint32(2))
    x0, x1 = _round4(x0, x1, _R0)
    x0, x1 = x0 + _KS0, x1 + (_KS1 + np.uint32(3))
    x0, x1 = _round4(x0, x1, _R1)
    x0, x1 = x0 + _KS1, x1 + (_KS2 + np.uint32(4))
    x0, x1 = _round4(x0, x1, _R0)
    x0, x1 = x0 + _KS2, x1 + (_KS0 + np.uint32(5))
    return x0 ^ x1


def _sample_block(logits_ref, idx_ref, x_ref):
    pid = pl.program_id(0)
    base = (pid * (ROWS * VOCAB)).astype(jnp.uint32)
    r = lax.broadcasted_iota(jnp.uint32, (ROWS, VOCAB), 0)
    c = lax.broadcasted_iota(jnp.uint32, (ROWS, VOCAB), 1)
    lin = base + r * np.uint32(VOCAB) + c

    bits = _threefry_fold(lin)
    f = lax.bitcast_convert_type(
        lax.shift_right_logical(bits, jnp.uint32(9)) | jnp.uint32(0x3F800000),
        jnp.float32,
    ) - jnp.float32(1.0)
    u = f * (jnp.float32(1.0) - _TINY) + _TINY
    u = jnp.maximum(_TINY, u)
    g = -jnp.log(-jnp.log(u))

    s = logits_ref[...] + g
    mx = jnp.max(s, axis=1, keepdims=True)
    col = lax.broadcasted_iota(jnp.int32, (ROWS, VOCAB), 1)
    idx = jnp.min(
        jnp.where(s == mx, col, jnp.int32(VOCAB)), axis=1, keepdims=True
    )
    idxf = idx.astype(jnp.float32)
    idx_ref[...] = idxf
    x_ref[...] = jnp.float32(-_A) + jnp.float32(_SCALE) * idxf


@functools.partial(jax.jit, static_argnums=())
def kernel(logits):
    n_rows = logits.shape[0]
    grid = (n_rows // ROWS,)
    idx, x = pl.pallas_call(
        _sample_block,
        grid=grid,
        in_specs=[pl.BlockSpec((ROWS, VOCAB), lambda i: (i, 0))],
        out_specs=[
            pl.BlockSpec((ROWS, 1), lambda i: (i, 0)),
            pl.BlockSpec((ROWS, 1), lambda i: (i, 0)),
        ],
        out_shape=[
            jax.ShapeDtypeStruct((n_rows, 1), jnp.float32),
            jax.ShapeDtypeStruct((n_rows, 1), jnp.float32),
        ],
    )(logits)
    return jnp.concatenate([idx, x], axis=-1)
